# baseline (speedup 1.0000x reference)
import functools

import jax
import jax.numpy as jnp
from jax import lax
from jax.experimental import pallas as pl
from jax.experimental.pallas import tpu as pltpu
import jax.experimental.pallas.tpu_sc as plsc

N = 10000
T = 4
C_IN = 128
C_OUT = 64
E = 160000

NC = 2
NS = 16
LANES = 16

HALF = (T * C_OUT) // NC

CHUNK = 80
CPS = 11
SUP = CHUNK * CPS
SUPS_PER_TILE = 12
EDGES_PER_TILE = SUP * SUPS_PER_TILE
E_PAD = EDGES_PER_TILE * NS
NSUP = NS * SUPS_PER_TILE

ZROW_TILE = 632
ZCHUNKS_LO = (80, 80, 80, 80, 80, 80, 80)
ZTAIL_LO = 72
CROWS = 624
CROWS_LAST = 640

BN = 1000

_SPLAT_DN = lax.GatherDimensionNumbers(
    offset_dims=(), collapsed_slice_dims=(0,), start_index_map=(0,))


def _linear_body(x_ref, w_ref, b_ref, y_ref):
    xblk = x_ref[...]
    y = jnp.dot(xblk, w_ref[...], preferred_element_type=jnp.float32)
    y_ref[0] = y + b_ref[...]


def _linear(x5, w2, b2):
    return pl.pallas_call(
        _linear_body,
        grid=(N // BN, NC),
        in_specs=[
            pl.BlockSpec((BN, 2 * C_IN), lambda nb, h: (nb, h)),
            pl.BlockSpec((2 * C_IN, HALF), lambda nb, h: (0, 0)),
            pl.BlockSpec((1, HALF), lambda nb, h: (0, 0)),
        ],
        out_specs=pl.BlockSpec((1, BN, HALF), lambda nb, h: (h, nb, 0)),
        out_shape=jax.ShapeDtypeStruct((NC, N, HALF), jnp.float32),
    )(x5, w2, b2)


_mesh = plsc.VectorSubcoreMesh(core_axis_name="c", subcore_axis_name="s")


@functools.partial(
    pl.kernel,
    out_type=jax.ShapeDtypeStruct((N, NC * HALF), jnp.float32),
    mesh=_mesh,
    scratch_types=[
        pltpu.VMEM((2, CPS, CHUNK), jnp.int32),
        pltpu.VMEM((2, CPS, CHUNK), jnp.int32),
        pltpu.VMEM((2, CPS, CHUNK), jnp.float32),
        pltpu.VMEM((LANES * LANES,), jnp.int32),
        pltpu.VMEM((CHUNK, HALF), jnp.float32),
        pltpu.VMEM((CHUNK, HALF), jnp.float32),
        pltpu.VMEM((CHUNK, HALF), jnp.float32),
        pltpu.VMEM_SHARED((N, HALF), jnp.float32),
        pltpu.SemaphoreType.DMA,
        pltpu.SemaphoreType.DMA,
        pltpu.SemaphoreType.DMA,
        pltpu.SemaphoreType.DMA,
        pltpu.SemaphoreType.DMA,
        pltpu.SemaphoreType.DMA,
        pltpu.SemaphoreType.DMA,
        pltpu.SemaphoreType.DMA,
    ],
)
def _spmm(y_hbm, src_hbm, dst_hbm, val_hbm, out_hbm,
          src_v, dst_v, val_v, sel_v, rows0, rows1, rows2, acc,
          sg0, sg1, sg2, ss0, ss1, ss2, si0, si1):
    c = lax.axis_index("c")
    s = lax.axis_index("s")
    rows = (rows0, rows1, rows2)
    sem_g = (sg0, sg1, sg2)
    sem_s = (ss0, ss1, ss2)
    sem_i = (si0, si1)

    for j in range(LANES):
        sel_v[pl.ds(j * LANES, LANES)] = jnp.full((LANES,), j, jnp.int32)

    zv = jnp.zeros((LANES,), jnp.float32)

    def zfill(r, _):
        for f in range(HALF // LANES):
            rows0[r, pl.ds(f * LANES, LANES)] = zv
        return 0

    lax.fori_loop(0, CHUNK, zfill, 0)
    zrow0 = s * ZROW_TILE

    @pl.when(s < NS - 1)
    def _():
        off = 0
        for nr in ZCHUNKS_LO:
            pltpu.sync_copy(rows0.at[pl.ds(0, nr)],
                            acc.at[pl.ds(zrow0 + off, nr)])
            off += nr
        pltpu.sync_copy(rows0.at[pl.ds(0, ZTAIL_LO)],
                        acc.at[pl.ds(zrow0 + 560, ZTAIL_LO)])

    @pl.when(s == NS - 1)
    def _():
        base = (NS - 1) * ZROW_TILE
        for q in range(6):
            pltpu.sync_copy(rows0.at[pl.ds(0, 80)],
                            acc.at[pl.ds(base + q * 80, 80)])
        pltpu.sync_copy(rows0.at[pl.ds(0, 40)],
                        acc.at[pl.ds(base + 480, 40)])

    plsc.subcore_barrier()

    offs = jnp.full((LANES,), c * N, jnp.int32)

    def stage(sup_id, bs):
        pltpu.async_copy(src_hbm.at[sup_id], src_v.at[bs], sem_i[bs])
        pltpu.async_copy(dst_hbm.at[sup_id], dst_v.at[bs], sem_i[bs])
        pltpu.async_copy(val_hbm.at[sup_id], val_v.at[bs], sem_i[bs])

    def stage_wait(bs):
        pltpu.make_async_copy(src_hbm.at[0], src_v.at[bs], sem_i[bs]).wait()
        pltpu.make_async_copy(dst_hbm.at[0], dst_v.at[bs], sem_i[bs]).wait()
        pltpu.make_async_copy(val_hbm.at[0], val_v.at[bs], sem_i[bs]).wait()

    def scale_chunk(b, k, buf):
        def group_body(g, _):
            vv = val_v[b, k, pl.ds(g * LANES, LANES)]
            for j in range(LANES):
                jidx = jnp.full((LANES, 1), j, jnp.int32)
                sp = lax.gather(vv, jidx, _SPLAT_DN, (1,),
                                mode=lax.GatherScatterMode.PROMISE_IN_BOUNDS)
                e = g * LANES + j
                for f in range(HALF // LANES):
                    sl = pl.ds(f * LANES, LANES)
                    buf[e, sl] = buf[e, sl] * sp
            return 0

        lax.fori_loop(0, CHUNK // LANES, group_body, 0)

    def process_super(m, b):
        nxt = jnp.minimum(m + 1, SUPS_PER_TILE - 1)

        @pl.when(b == 0)
        def _():
            stage_wait(0)
            stage(s * SUPS_PER_TILE + nxt, 1)

        @pl.when(b == 1)
        def _():
            stage_wait(1)
            stage(s * SUPS_PER_TILE + nxt, 0)

        def add_offs(r, _):
            for f in range(CHUNK // LANES):
                sl = pl.ds(f * LANES, LANES)
                src_v[b, r, sl] = src_v[b, r, sl] + offs
            return 0

        lax.fori_loop(0, CPS, add_offs, 0)

        gathers = [None] * CPS
        scatters = [None] * CPS
        for k in range(2):
            gathers[k] = pltpu.async_copy(
                y_hbm.at[src_v.at[b, k]], rows[k % 3], sem_g[k % 3])
        for k in range(CPS):
            p = k % 3
            gathers[k].wait()
            scale_chunk(b, k, rows[p])
            scatters[k] = pltpu.async_copy(
                rows[p], acc.at[dst_v.at[b, k]], sem_s[p], add=True)
            if k + 2 < CPS:
                q = (k + 2) % 3
                if k - 1 >= 0:
                    scatters[k - 1].wait()
                gathers[k + 2] = pltpu.async_copy(
                    y_hbm.at[src_v.at[b, k + 2]], rows[q], sem_g[q])
        for k in range(CPS - 3, CPS):
            scatters[k].wait()

    stage(s * SUPS_PER_TILE, 0)

    def super_body(m, _):
        process_super(m, m % 2)
        return 0

    lax.fori_loop(0, SUPS_PER_TILE, super_body, 0)
    stage_wait(0)
    plsc.subcore_barrier()

    col0 = pl.multiple_of(c * HALF, HALF)

    @pl.when(s < NS - 1)
    def _():
        r = s * CROWS
        pltpu.sync_copy(acc.at[pl.ds(r, CROWS)],
                        out_hbm.at[pl.ds(r, CROWS), pl.ds(col0, HALF)])

    @pl.when(s == NS - 1)
    def _():
        r = (NS - 1) * CROWS
        pltpu.sync_copy(acc.at[pl.ds(r, CROWS_LAST)],
                        out_hbm.at[pl.ds(r, CROWS_LAST), pl.ds(col0, HALF)])


def kernel(x, adj_indices, adj_values, W, b):
    x5 = x.reshape(N, T * C_IN)
    w2 = jnp.zeros((2 * C_IN, HALF), jnp.float32)
    w2 = w2.at[:C_IN, :C_OUT].set(W).at[C_IN:, C_OUT:].set(W)
    b2 = jnp.concatenate([b, b]).reshape(1, HALF)

    y2 = _linear(x5, w2, b2)

    pad = E_PAD - E
    dst = jnp.pad(adj_indices[0], (0, pad)).reshape(NSUP, CPS, CHUNK)
    src = jnp.pad(adj_indices[1], (0, pad)).reshape(NSUP, CPS, CHUNK)
    val = jnp.pad(adj_values, (0, pad)).reshape(NSUP, CPS, CHUNK)

    out2 = _spmm(y2.reshape(NC * N, HALF), src, dst, val)
    return out2.reshape(1, N, T, C_OUT)


# speedup vs baseline: 2.4579x; 2.4579x over previous
import functools

import jax
import jax.numpy as jnp
from jax import lax
from jax.experimental import pallas as pl
from jax.experimental.pallas import tpu as pltpu
import jax.experimental.pallas.tpu_sc as plsc

N = 10000
T = 4
C_IN = 128
C_OUT = 64
E = 160000

NC = 2
NS = 16
LANES = 16

HALF = (T * C_OUT) // NC

CHUNK = 80
CPS = 7
SUP = CHUNK * CPS
SUPS_PER_TILE = 18
EDGES_PER_TILE = SUP * SUPS_PER_TILE
E_PAD = EDGES_PER_TILE * NS
NSUP = NS * SUPS_PER_TILE

ZROW_TILE = 632
CROWS = 624
CROWS_LAST = 640

BN = 1000

_SPLAT_DN = lax.GatherDimensionNumbers(
    offset_dims=(), collapsed_slice_dims=(0,), start_index_map=(0,))


def _linear_body(x_ref, w_ref, b_ref, y_ref):
    xblk = x_ref[...]
    y = jnp.dot(xblk, w_ref[...], preferred_element_type=jnp.float32)
    y_ref[0] = y + b_ref[...]


def _linear(x5, w2, b2):
    return pl.pallas_call(
        _linear_body,
        grid=(N // BN, NC),
        in_specs=[
            pl.BlockSpec((BN, 2 * C_IN), lambda nb, h: (nb, h)),
            pl.BlockSpec((2 * C_IN, HALF), lambda nb, h: (0, 0)),
            pl.BlockSpec((1, HALF), lambda nb, h: (0, 0)),
        ],
        out_specs=pl.BlockSpec((1, BN, HALF), lambda nb, h: (h, nb, 0)),
        out_shape=jax.ShapeDtypeStruct((NC, N, HALF), jnp.float32),
    )(x5, w2, b2)


_mesh = plsc.VectorSubcoreMesh(core_axis_name="c", subcore_axis_name="s")


@functools.partial(
    pl.kernel,
    out_type=jax.ShapeDtypeStruct((N, NC * HALF), jnp.float32),
    mesh=_mesh,
    scratch_types=[
        pltpu.VMEM((2, CPS, CHUNK), jnp.int32),
        pltpu.VMEM((2, CPS, CHUNK), jnp.int32),
        pltpu.VMEM((2, CPS, CHUNK), jnp.float32),
        pltpu.VMEM((CHUNK, HALF), jnp.float32),
        pltpu.VMEM((CHUNK, HALF), jnp.float32),
        pltpu.VMEM((CHUNK, HALF), jnp.float32),
        pltpu.VMEM_SHARED((N, HALF), jnp.float32),
        pltpu.SemaphoreType.DMA,
        pltpu.SemaphoreType.DMA,
        pltpu.SemaphoreType.DMA,
        pltpu.SemaphoreType.DMA,
        pltpu.SemaphoreType.DMA,
        pltpu.SemaphoreType.DMA,
        pltpu.SemaphoreType.DMA,
        pltpu.SemaphoreType.DMA,
    ],
)
def _spmm(y_hbm, src_hbm, dst_hbm, val_hbm, out_hbm,
          src_v, dst_v, val_v, rows0, rows1, rows2, acc,
          sg0, sg1, sg2, ss0, ss1, ss2, si0, si1):
    c = lax.axis_index("c")
    s = lax.axis_index("s")
    rows = (rows0, rows1, rows2)
    sem_g = (sg0, sg1, sg2)
    sem_s = (ss0, ss1, ss2)
    sem_i = (si0, si1)

    zv = jnp.zeros((LANES,), jnp.float32)

    def zfill(r, _):
        for f in range(HALF // LANES):
            rows0[r, pl.ds(f * LANES, LANES)] = zv
        return 0

    lax.fori_loop(0, CHUNK, zfill, 0)

    @pl.when(s < NS - 1)
    def _():
        zrow0 = s * ZROW_TILE
        for q in range(7):
            pltpu.sync_copy(rows0.at[pl.ds(0, 80)],
                            acc.at[pl.ds(zrow0 + q * 80, 80)])
        pltpu.sync_copy(rows0.at[pl.ds(0, 72)],
                        acc.at[pl.ds(zrow0 + 560, 72)])

    @pl.when(s == NS - 1)
    def _():
        base = (NS - 1) * ZROW_TILE
        for q in range(6):
            pltpu.sync_copy(rows0.at[pl.ds(0, 80)],
                            acc.at[pl.ds(base + q * 80, 80)])
        pltpu.sync_copy(rows0.at[pl.ds(0, 40)],
                        acc.at[pl.ds(base + 480, 40)])

    plsc.subcore_barrier()

    offs = jnp.full((LANES,), c * N, jnp.int32)

    def stage(sup_id, bs):
        pltpu.async_copy(src_hbm.at[sup_id], src_v.at[bs], sem_i[bs])
        pltpu.async_copy(dst_hbm.at[sup_id], dst_v.at[bs], sem_i[bs])
        pltpu.async_copy(val_hbm.at[sup_id], val_v.at[bs], sem_i[bs])

    def stage_wait(bs):
        pltpu.make_async_copy(src_hbm.at[0], src_v.at[bs], sem_i[bs]).wait()
        pltpu.make_async_copy(dst_hbm.at[0], dst_v.at[bs], sem_i[bs]).wait()
        pltpu.make_async_copy(val_hbm.at[0], val_v.at[bs], sem_i[bs]).wait()

    def scale_chunk(bs, k, buf):
        def group_body(g, _):
            vv = val_v[bs, k, pl.ds(g * LANES, LANES)]
            for j in range(LANES):
                jidx = jnp.full((LANES, 1), j, jnp.int32)
                sp = lax.gather(vv, jidx, _SPLAT_DN, (1,),
                                mode=lax.GatherScatterMode.PROMISE_IN_BOUNDS)
                e = g * LANES + j
                for f in range(HALF // LANES):
                    sl = pl.ds(f * LANES, LANES)
                    buf[e, sl] = buf[e, sl] * sp
            return 0

        lax.fori_loop(0, CHUNK // LANES, group_body, 0)

    def process_super(sup_id, nxt_id, bs):
        stage_wait(bs)

        def add_offs(r, _):
            for f in range(CHUNK // LANES):
                sl = pl.ds(f * LANES, LANES)
                src_v[bs, r, sl] = src_v[bs, r, sl] + offs
            return 0

        lax.fori_loop(0, CPS, add_offs, 0)
        stage(nxt_id, 1 - bs)

        gathers = [None] * CPS
        scatters = [None] * CPS
        for k in range(2):
            gathers[k] = pltpu.async_copy(
                y_hbm.at[src_v.at[bs, k]], rows[k % 3], sem_g[k % 3])
        for k in range(CPS):
            p = k % 3
            gathers[k].wait()
            scale_chunk(bs, k, rows[p])
            scatters[k] = pltpu.async_copy(
                rows[p], acc.at[dst_v.at[bs, k]], sem_s[p], add=True)
            if k + 2 < CPS:
                q = (k + 2) % 3
                if k - 1 >= 0:
                    scatters[k - 1].wait()
                gathers[k + 2] = pltpu.async_copy(
                    y_hbm.at[src_v.at[bs, k + 2]], rows[q], sem_g[q])
        for k in range(CPS - 3, CPS):
            scatters[k].wait()

    sup0 = s * SUPS_PER_TILE
    stage(sup0, 0)

    def super_pair(i, _):
        m = sup0 + 2 * i
        process_super(m, m + 1, 0)
        nxt2 = jnp.minimum(m + 2, sup0 + SUPS_PER_TILE - 1)
        process_super(m + 1, nxt2, 1)
        return 0

    lax.fori_loop(0, SUPS_PER_TILE // 2, super_pair, 0)
    stage_wait(0)
    plsc.subcore_barrier()

    col0 = pl.multiple_of(c * HALF, HALF)

    @pl.when(s < NS - 1)
    def _():
        r = s * CROWS
        pltpu.sync_copy(acc.at[pl.ds(r, CROWS)],
                        out_hbm.at[pl.ds(r, CROWS), pl.ds(col0, HALF)])

    @pl.when(s == NS - 1)
    def _():
        r = (NS - 1) * CROWS
        pltpu.sync_copy(acc.at[pl.ds(r, CROWS_LAST)],
                        out_hbm.at[pl.ds(r, CROWS_LAST), pl.ds(col0, HALF)])


def kernel(x, adj_indices, adj_values, W, b):
    x5 = x.reshape(N, T * C_IN)
    w2 = jnp.zeros((2 * C_IN, HALF), jnp.float32)
    w2 = w2.at[:C_IN, :C_OUT].set(W).at[C_IN:, C_OUT:].set(W)
    b2 = jnp.concatenate([b, b]).reshape(1, HALF)

    y2 = _linear(x5, w2, b2)

    pad = E_PAD - E
    dst = jnp.pad(adj_indices[0], (0, pad)).reshape(NSUP, CPS, CHUNK)
    src = jnp.pad(adj_indices[1], (0, pad)).reshape(NSUP, CPS, CHUNK)
    val = jnp.pad(adj_values, (0, pad)).reshape(NSUP, CPS, CHUNK)

    out2 = _spmm(y2.reshape(NC * N, HALF), src, dst, val)
    return out2.reshape(1, N, T, C_OUT)


# speedup vs baseline: 2.4887x; 1.0125x over previous
import functools

import jax
import jax.numpy as jnp
from jax import lax
from jax.experimental import pallas as pl
from jax.experimental.pallas import tpu as pltpu
import jax.experimental.pallas.tpu_sc as plsc

N = 10000
T = 4
C_IN = 128
C_OUT = 64
E = 160000

NC = 2
NS = 16
LANES = 16

HALF = (T * C_OUT) // NC

CHUNK = 80
CPS = 14
SUP = CHUNK * CPS
SUPS_PER_TILE = 9
EDGES_PER_TILE = SUP * SUPS_PER_TILE
E_PAD = EDGES_PER_TILE * NS
NSUP = NS * SUPS_PER_TILE

ZROW_TILE = 632
CROWS = 624
CROWS_LAST = 640

BN = 1000

_SPLAT_DN = lax.GatherDimensionNumbers(
    offset_dims=(), collapsed_slice_dims=(0,), start_index_map=(0,))


def _linear_body(x_ref, w_ref, b_ref, y_ref):
    xblk = x_ref[...]
    y = jnp.dot(xblk, w_ref[...], preferred_element_type=jnp.float32)
    y_ref[0] = y + b_ref[...]


def _linear(x5, w2, b2):
    return pl.pallas_call(
        _linear_body,
        grid=(N // BN, NC),
        in_specs=[
            pl.BlockSpec((BN, 2 * C_IN), lambda nb, h: (nb, h)),
            pl.BlockSpec((2 * C_IN, HALF), lambda nb, h: (0, 0)),
            pl.BlockSpec((1, HALF), lambda nb, h: (0, 0)),
        ],
        out_specs=pl.BlockSpec((1, BN, HALF), lambda nb, h: (h, nb, 0)),
        out_shape=jax.ShapeDtypeStruct((NC, N, HALF), jnp.float32),
    )(x5, w2, b2)


_mesh = plsc.VectorSubcoreMesh(core_axis_name="c", subcore_axis_name="s")


@functools.partial(
    pl.kernel,
    out_type=jax.ShapeDtypeStruct((N, NC * HALF), jnp.float32),
    mesh=_mesh,
    scratch_types=[
        pltpu.VMEM((CPS, CHUNK), jnp.int32),
        pltpu.VMEM((CPS, CHUNK), jnp.int32),
        pltpu.VMEM((CPS, CHUNK), jnp.float32),
        pltpu.VMEM((CHUNK, HALF), jnp.float32),
        pltpu.VMEM((CHUNK, HALF), jnp.float32),
        pltpu.VMEM((CHUNK, HALF), jnp.float32),
        pltpu.VMEM_SHARED((N, HALF), jnp.float32),
        pltpu.SemaphoreType.DMA,
        pltpu.SemaphoreType.DMA,
        pltpu.SemaphoreType.DMA,
        pltpu.SemaphoreType.DMA,
        pltpu.SemaphoreType.DMA,
        pltpu.SemaphoreType.DMA,
    ],
)
def _spmm(y_hbm, src_hbm, dst_hbm, val_hbm, out_hbm,
          src_v, dst_v, val_v, rows0, rows1, rows2, acc,
          sg0, sg1, sg2, ss0, ss1, ss2):
    c = lax.axis_index("c")
    s = lax.axis_index("s")
    rows = (rows0, rows1, rows2)
    sem_g = (sg0, sg1, sg2)
    sem_s = (ss0, ss1, ss2)

    zv = jnp.zeros((LANES,), jnp.float32)

    def zfill(r, _):
        for f in range(HALF // LANES):
            rows0[r, pl.ds(f * LANES, LANES)] = zv
        return 0

    lax.fori_loop(0, CHUNK, zfill, 0)

    @pl.when(s < NS - 1)
    def _():
        zrow0 = s * ZROW_TILE
        for q in range(7):
            pltpu.sync_copy(rows0.at[pl.ds(0, 80)],
                            acc.at[pl.ds(zrow0 + q * 80, 80)])
        pltpu.sync_copy(rows0.at[pl.ds(0, 72)],
                        acc.at[pl.ds(zrow0 + 560, 72)])

    @pl.when(s == NS - 1)
    def _():
        base = (NS - 1) * ZROW_TILE
        for q in range(6):
            pltpu.sync_copy(rows0.at[pl.ds(0, 80)],
                            acc.at[pl.ds(base + q * 80, 80)])
        pltpu.sync_copy(rows0.at[pl.ds(0, 40)],
                        acc.at[pl.ds(base + 480, 40)])

    plsc.subcore_barrier()

    offs = jnp.full((LANES,), c * N, jnp.int32)

    def scale_chunk(k, buf):
        def group_body(g, _):
            vv = val_v[k, pl.ds(g * LANES, LANES)]
            for j in range(LANES):
                jidx = jnp.full((LANES, 1), j, jnp.int32)
                sp = lax.gather(vv, jidx, _SPLAT_DN, (1,),
                                mode=lax.GatherScatterMode.PROMISE_IN_BOUNDS)
                e = g * LANES + j
                for f in range(HALF // LANES):
                    sl = pl.ds(f * LANES, LANES)
                    buf[e, sl] = buf[e, sl] * sp
            return 0

        lax.fori_loop(0, CHUNK // LANES, group_body, 0)

    def super_body(m, _):
        sup_id = s * SUPS_PER_TILE + m
        pltpu.sync_copy(src_hbm.at[sup_id], src_v)
        pltpu.sync_copy(dst_hbm.at[sup_id], dst_v)
        pltpu.sync_copy(val_hbm.at[sup_id], val_v)

        def add_offs(r, _):
            for f in range(CHUNK // LANES):
                sl = pl.ds(f * LANES, LANES)
                src_v[r, sl] = src_v[r, sl] + offs
            return 0

        lax.fori_loop(0, CPS, add_offs, 0)

        gathers = [None] * CPS
        scatters = [None] * CPS
        for k in range(2):
            gathers[k] = pltpu.async_copy(
                y_hbm.at[src_v.at[k]], rows[k % 3], sem_g[k % 3])
        for k in range(CPS):
            p = k % 3
            gathers[k].wait()
            scale_chunk(k, rows[p])
            scatters[k] = pltpu.async_copy(
                rows[p], acc.at[dst_v.at[k]], sem_s[p], add=True)
            if k + 2 < CPS:
                q = (k + 2) % 3
                if k - 1 >= 0:
                    scatters[k - 1].wait()
                gathers[k + 2] = pltpu.async_copy(
                    y_hbm.at[src_v.at[k + 2]], rows[q], sem_g[q])
        for k in range(CPS - 3, CPS):
            scatters[k].wait()
        return 0

    lax.fori_loop(0, SUPS_PER_TILE, super_body, 0)
    plsc.subcore_barrier()

    col0 = pl.multiple_of(c * HALF, HALF)

    @pl.when(s < NS - 1)
    def _():
        r = s * CROWS
        pltpu.sync_copy(acc.at[pl.ds(r, CROWS)],
                        out_hbm.at[pl.ds(r, CROWS), pl.ds(col0, HALF)])

    @pl.when(s == NS - 1)
    def _():
        r = (NS - 1) * CROWS
        pltpu.sync_copy(acc.at[pl.ds(r, CROWS_LAST)],
                        out_hbm.at[pl.ds(r, CROWS_LAST), pl.ds(col0, HALF)])


def kernel(x, adj_indices, adj_values, W, b):
    x5 = x.reshape(N, T * C_IN)
    w2 = jnp.zeros((2 * C_IN, HALF), jnp.float32)
    w2 = w2.at[:C_IN, :C_OUT].set(W).at[C_IN:, C_OUT:].set(W)
    b2 = jnp.concatenate([b, b]).reshape(1, HALF)

    y2 = _linear(x5, w2, b2)

    pad = E_PAD - E
    dst = jnp.pad(adj_indices[0], (0, pad)).reshape(NSUP, CPS, CHUNK)
    src = jnp.pad(adj_indices[1], (0, pad)).reshape(NSUP, CPS, CHUNK)
    val = jnp.pad(adj_values, (0, pad)).reshape(NSUP, CPS, CHUNK)

    out2 = _spmm(y2.reshape(NC * N, HALF), src, dst, val)
    return out2.reshape(1, N, T, C_OUT)


# speedup vs baseline: 3.1342x; 1.2594x over previous
import functools

import jax
import jax.numpy as jnp
from jax import lax
from jax.experimental import pallas as pl
from jax.experimental.pallas import tpu as pltpu
import jax.experimental.pallas.tpu_sc as plsc

N = 10000
T = 4
C_IN = 128
C_OUT = 64
E = 160000

NC = 2
NS = 16
LANES = 16

HALF = (T * C_OUT) // NC

CHUNK = 80
CPS = 14
SUP = CHUNK * CPS
SUPS_PER_TILE = 9
EDGES_PER_TILE = SUP * SUPS_PER_TILE
E_PAD = EDGES_PER_TILE * NS
NSUP = NS * SUPS_PER_TILE

ZROW_TILE = 632
CROWS = 624
CROWS_LAST = 640

BN = 1000

_SPLAT_DN = lax.GatherDimensionNumbers(
    offset_dims=(), collapsed_slice_dims=(0,), start_index_map=(0,))


def _linear_body(x_ref, w_ref, b_ref, y_ref):
    xblk = x_ref[...]
    y = jnp.dot(xblk, w_ref[...], preferred_element_type=jnp.float32)
    y_ref[0] = y + b_ref[...]


def _linear(x5, w2, b2):
    return pl.pallas_call(
        _linear_body,
        grid=(N // BN, NC),
        in_specs=[
            pl.BlockSpec((BN, 2 * C_IN), lambda nb, h: (nb, h)),
            pl.BlockSpec((2 * C_IN, HALF), lambda nb, h: (0, 0)),
            pl.BlockSpec((1, HALF), lambda nb, h: (0, 0)),
        ],
        out_specs=pl.BlockSpec((1, BN, HALF), lambda nb, h: (h, nb, 0)),
        out_shape=jax.ShapeDtypeStruct((NC, N, HALF), jnp.float32),
    )(x5, w2, b2)


_mesh = plsc.VectorSubcoreMesh(core_axis_name="c", subcore_axis_name="s")


@functools.partial(
    pl.kernel,
    out_type=jax.ShapeDtypeStruct((N, NC * HALF), jnp.float32),
    mesh=_mesh,
    scratch_types=[
        pltpu.VMEM((SUP,), jnp.int32),
        pltpu.VMEM((SUP,), jnp.int32),
        pltpu.VMEM((SUP,), jnp.float32),
        pltpu.VMEM((CHUNK, HALF), jnp.float32),
        pltpu.VMEM((CHUNK, HALF), jnp.float32),
        pltpu.VMEM((CHUNK, HALF), jnp.float32),
        pltpu.VMEM_SHARED((N, HALF), jnp.float32),
        pltpu.SemaphoreType.DMA,
        pltpu.SemaphoreType.DMA,
        pltpu.SemaphoreType.DMA,
        pltpu.SemaphoreType.DMA,
        pltpu.SemaphoreType.DMA,
        pltpu.SemaphoreType.DMA,
    ],
)
def _spmm(y_hbm, src_hbm, dst_hbm, val_hbm, out_hbm,
          src_v, dst_v, val_v, rows0, rows1, rows2, acc,
          sg0, sg1, sg2, ss0, ss1, ss2):
    c = lax.axis_index("c")
    s = lax.axis_index("s")
    rows = (rows0, rows1, rows2)
    sem_g = (sg0, sg1, sg2)
    sem_s = (ss0, ss1, ss2)

    zv = jnp.zeros((LANES,), jnp.float32)

    def zfill(r, _):
        for f in range(HALF // LANES):
            rows0[r, pl.ds(f * LANES, LANES)] = zv
        return 0

    lax.fori_loop(0, CHUNK, zfill, 0)

    @pl.when(s < NS - 1)
    def _():
        zrow0 = s * ZROW_TILE
        for q in range(7):
            pltpu.sync_copy(rows0.at[pl.ds(0, 80)],
                            acc.at[pl.ds(zrow0 + q * 80, 80)])
        pltpu.sync_copy(rows0.at[pl.ds(0, 72)],
                        acc.at[pl.ds(zrow0 + 560, 72)])

    @pl.when(s == NS - 1)
    def _():
        base = (NS - 1) * ZROW_TILE
        for q in range(6):
            pltpu.sync_copy(rows0.at[pl.ds(0, 80)],
                            acc.at[pl.ds(base + q * 80, 80)])
        pltpu.sync_copy(rows0.at[pl.ds(0, 40)],
                        acc.at[pl.ds(base + 480, 40)])

    plsc.subcore_barrier()

    offs = jnp.full((LANES,), c * N, jnp.int32)

    def scale_chunk(k, buf):
        def group_body(g, _):
            vv = val_v[pl.ds(k * CHUNK + g * LANES, LANES)]
            for j in range(LANES):
                jidx = jnp.full((LANES, 1), j, jnp.int32)
                sp = lax.gather(vv, jidx, _SPLAT_DN, (1,),
                                mode=lax.GatherScatterMode.PROMISE_IN_BOUNDS)
                e = g * LANES + j
                for f in range(HALF // LANES):
                    sl = pl.ds(f * LANES, LANES)
                    buf[e, sl] = buf[e, sl] * sp
            return 0

        lax.fori_loop(0, CHUNK // LANES, group_body, 0)

    def super_body(m, _):
        base = s * (E // NS) + jnp.minimum(m * SUP, E // NS - SUP)
        pltpu.sync_copy(src_hbm.at[pl.ds(base, SUP)], src_v)
        pltpu.sync_copy(dst_hbm.at[pl.ds(base, SUP)], dst_v)
        pltpu.sync_copy(val_hbm.at[pl.ds(base, SUP)], val_v)

        @pl.when(m == SUPS_PER_TILE - 1)
        def _():
            for f in range(CHUNK // LANES):
                val_v[pl.ds(f * LANES, LANES)] = zv

        def add_offs(r, _):
            sl = pl.ds(r * LANES, LANES)
            src_v[sl] = src_v[sl] + offs
            return 0

        lax.fori_loop(0, SUP // LANES, add_offs, 0)

        gathers = [None] * CPS
        scatters = [None] * CPS
        for k in range(2):
            gathers[k] = pltpu.async_copy(
                y_hbm.at[src_v.at[pl.ds(k * CHUNK, CHUNK)]],
                rows[k % 3], sem_g[k % 3])
        for k in range(CPS):
            p = k % 3
            gathers[k].wait()
            scale_chunk(k, rows[p])
            scatters[k] = pltpu.async_copy(
                rows[p], acc.at[dst_v.at[pl.ds(k * CHUNK, CHUNK)]],
                sem_s[p], add=True)
            if k + 2 < CPS:
                q = (k + 2) % 3
                if k - 1 >= 0:
                    scatters[k - 1].wait()
                gathers[k + 2] = pltpu.async_copy(
                    y_hbm.at[src_v.at[pl.ds((k + 2) * CHUNK, CHUNK)]],
                    rows[q], sem_g[q])
        for k in range(CPS - 3, CPS):
            scatters[k].wait()
        return 0

    lax.fori_loop(0, SUPS_PER_TILE, super_body, 0)
    plsc.subcore_barrier()

    col0 = pl.multiple_of(c * HALF, HALF)

    @pl.when(s < NS - 1)
    def _():
        r = s * CROWS
        pltpu.sync_copy(acc.at[pl.ds(r, CROWS)],
                        out_hbm.at[pl.ds(r, CROWS), pl.ds(col0, HALF)])

    @pl.when(s == NS - 1)
    def _():
        r = (NS - 1) * CROWS
        pltpu.sync_copy(acc.at[pl.ds(r, CROWS_LAST)],
                        out_hbm.at[pl.ds(r, CROWS_LAST), pl.ds(col0, HALF)])


def kernel(x, adj_indices, adj_values, W, b):
    x5 = x.reshape(N, T * C_IN)
    w2 = jnp.zeros((2 * C_IN, HALF), jnp.float32)
    w2 = w2.at[:C_IN, :C_OUT].set(W).at[C_IN:, C_OUT:].set(W)
    b2 = jnp.concatenate([b, b]).reshape(1, HALF)

    y2 = _linear(x5, w2, b2)

    out2 = _spmm(y2.reshape(NC * N, HALF), adj_indices[1], adj_indices[0],
                 adj_values)
    return out2.reshape(1, N, T, C_OUT)


# speedup vs baseline: 3.1424x; 1.0026x over previous
import functools

import jax
import jax.numpy as jnp
from jax import lax
from jax.experimental import pallas as pl
from jax.experimental.pallas import tpu as pltpu
import jax.experimental.pallas.tpu_sc as plsc

N = 10000
T = 4
C_IN = 128
C_OUT = 64
E = 160000

NC = 2
NS = 16
LANES = 16

HALF = (T * C_OUT) // NC

CHUNK = 80
CPS = 14
SUP = CHUNK * CPS
SUPS_PER_TILE = 9
EDGES_PER_TILE = SUP * SUPS_PER_TILE
E_PAD = EDGES_PER_TILE * NS
NSUP = NS * SUPS_PER_TILE

ZROW_TILE = 632
CROWS = 624
CROWS_LAST = 640

BN = 1000

_SPLAT_DN = lax.GatherDimensionNumbers(
    offset_dims=(), collapsed_slice_dims=(0,), start_index_map=(0,))


def _linear_body(x_ref, w_ref, b_ref, y_ref):
    xblk = x_ref[...]
    y = jnp.dot(xblk, w_ref[...], preferred_element_type=jnp.float32)
    y_ref[...] = y + b_ref[...]


def _linear(x5, w2, b2):
    return pl.pallas_call(
        _linear_body,
        grid=(N // BN, NC),
        in_specs=[
            pl.BlockSpec((BN, 2 * C_IN), lambda nb, h: (nb, h)),
            pl.BlockSpec((2 * C_IN, HALF), lambda nb, h: (0, 0)),
            pl.BlockSpec((1, HALF), lambda nb, h: (0, 0)),
        ],
        out_specs=pl.BlockSpec((BN, HALF), lambda nb, h: (h * (N // BN) + nb, 0)),
        out_shape=jax.ShapeDtypeStruct((NC * N, HALF), jnp.float32),
    )(x5, w2, b2)


_mesh = plsc.VectorSubcoreMesh(core_axis_name="c", subcore_axis_name="s")


@functools.partial(
    pl.kernel,
    out_type=jax.ShapeDtypeStruct((N, NC * HALF), jnp.float32),
    mesh=_mesh,
    scratch_types=[
        pltpu.VMEM((SUP,), jnp.int32),
        pltpu.VMEM((SUP,), jnp.int32),
        pltpu.VMEM((SUP,), jnp.float32),
        pltpu.VMEM((CHUNK, HALF), jnp.float32),
        pltpu.VMEM((CHUNK, HALF), jnp.float32),
        pltpu.VMEM((CHUNK, HALF), jnp.float32),
        pltpu.VMEM_SHARED((N, HALF), jnp.float32),
        pltpu.SemaphoreType.DMA,
        pltpu.SemaphoreType.DMA,
        pltpu.SemaphoreType.DMA,
        pltpu.SemaphoreType.DMA,
        pltpu.SemaphoreType.DMA,
        pltpu.SemaphoreType.DMA,
    ],
)
def _spmm(y_hbm, src_hbm, dst_hbm, val_hbm, out_hbm,
          src_v, dst_v, val_v, rows0, rows1, rows2, acc,
          sg0, sg1, sg2, ss0, ss1, ss2):
    c = lax.axis_index("c")
    s = lax.axis_index("s")
    rows = (rows0, rows1, rows2)
    sem_g = (sg0, sg1, sg2)
    sem_s = (ss0, ss1, ss2)

    zv = jnp.zeros((LANES,), jnp.float32)

    def zfill(r, _):
        for f in range(HALF // LANES):
            rows0[r, pl.ds(f * LANES, LANES)] = zv
        return 0

    lax.fori_loop(0, CHUNK, zfill, 0)

    @pl.when(s < NS - 1)
    def _():
        zrow0 = s * ZROW_TILE
        for q in range(7):
            pltpu.sync_copy(rows0.at[pl.ds(0, 80)],
                            acc.at[pl.ds(zrow0 + q * 80, 80)])
        pltpu.sync_copy(rows0.at[pl.ds(0, 72)],
                        acc.at[pl.ds(zrow0 + 560, 72)])

    @pl.when(s == NS - 1)
    def _():
        base = (NS - 1) * ZROW_TILE
        for q in range(6):
            pltpu.sync_copy(rows0.at[pl.ds(0, 80)],
                            acc.at[pl.ds(base + q * 80, 80)])
        pltpu.sync_copy(rows0.at[pl.ds(0, 40)],
                        acc.at[pl.ds(base + 480, 40)])

    plsc.subcore_barrier()

    offs = jnp.full((LANES,), c * N, jnp.int32)

    def scale_chunk(k, buf):
        def group_body(g, _):
            vv = val_v[pl.ds(k * CHUNK + g * LANES, LANES)]
            for j in range(LANES):
                jidx = jnp.full((LANES, 1), j, jnp.int32)
                sp = lax.gather(vv, jidx, _SPLAT_DN, (1,),
                                mode=lax.GatherScatterMode.PROMISE_IN_BOUNDS)
                e = g * LANES + j
                for f in range(HALF // LANES):
                    sl = pl.ds(f * LANES, LANES)
                    buf[e, sl] = buf[e, sl] * sp
            return 0

        lax.fori_loop(0, CHUNK // LANES, group_body, 0)

    def super_body(m, _):
        base = s * (E // NS) + jnp.minimum(m * SUP, E // NS - SUP)
        pltpu.sync_copy(src_hbm.at[pl.ds(base, SUP)], src_v)
        pltpu.sync_copy(dst_hbm.at[pl.ds(base, SUP)], dst_v)
        pltpu.sync_copy(val_hbm.at[pl.ds(base, SUP)], val_v)

        @pl.when(m == SUPS_PER_TILE - 1)
        def _():
            for f in range(CHUNK // LANES):
                val_v[pl.ds(f * LANES, LANES)] = zv

        def add_offs(r, _):
            sl = pl.ds(r * LANES, LANES)
            src_v[sl] = src_v[sl] + offs
            return 0

        lax.fori_loop(0, SUP // LANES, add_offs, 0)

        gathers = [None] * CPS
        scatters = [None] * CPS
        for k in range(2):
            gathers[k] = pltpu.async_copy(
                y_hbm.at[src_v.at[pl.ds(k * CHUNK, CHUNK)]],
                rows[k % 3], sem_g[k % 3])
        for k in range(CPS):
            p = k % 3
            gathers[k].wait()
            scale_chunk(k, rows[p])
            scatters[k] = pltpu.async_copy(
                rows[p], acc.at[dst_v.at[pl.ds(k * CHUNK, CHUNK)]],
                sem_s[p], add=True)
            if k + 2 < CPS:
                q = (k + 2) % 3
                if k - 1 >= 0:
                    scatters[k - 1].wait()
                gathers[k + 2] = pltpu.async_copy(
                    y_hbm.at[src_v.at[pl.ds((k + 2) * CHUNK, CHUNK)]],
                    rows[q], sem_g[q])
        for k in range(CPS - 3, CPS):
            scatters[k].wait()
        return 0

    lax.fori_loop(0, SUPS_PER_TILE, super_body, 0)
    plsc.subcore_barrier()

    col0 = pl.multiple_of(c * HALF, HALF)

    @pl.when(s < NS - 1)
    def _():
        r = s * CROWS
        pltpu.sync_copy(acc.at[pl.ds(r, CROWS)],
                        out_hbm.at[pl.ds(r, CROWS), pl.ds(col0, HALF)])

    @pl.when(s == NS - 1)
    def _():
        r = (NS - 1) * CROWS
        pltpu.sync_copy(acc.at[pl.ds(r, CROWS_LAST)],
                        out_hbm.at[pl.ds(r, CROWS_LAST), pl.ds(col0, HALF)])


def kernel(x, adj_indices, adj_values, W, b):
    x5 = x.reshape(N, T * C_IN)
    w2 = jnp.zeros((2 * C_IN, HALF), jnp.float32)
    w2 = w2.at[:C_IN, :C_OUT].set(W).at[C_IN:, C_OUT:].set(W)
    b2 = jnp.concatenate([b, b]).reshape(1, HALF)

    y2 = _linear(x5, w2, b2)

    out2 = _spmm(y2, adj_indices[1], adj_indices[0], adj_values)
    return out2.reshape(1, N, T, C_OUT)
